# baseline (speedup 1.0000x reference)
import jax
import jax.numpy as jnp
from jax import lax
from jax.experimental import pallas as pl
from jax.experimental.pallas import tpu as pltpu
from jax.experimental.pallas import tpu_sc as plsc

N = 10000
E = 320000
D = 128
H = 32
G = 64

NC = 2
NS = 16
NW = NC * NS
EPW = E // NW
EB = 500
NCHUNK = EPW // EB
NP = 10240
RPT = NP // NS

def _deg_body(er_hbm, zeros_hbm, ones_hbm, out_hbm, deg_sh, dst2, ones_v):
    cid = lax.axis_index("c")
    sid = lax.axis_index("s")
    wid = sid * NC + cid
    pltpu.sync_copy(zeros_hbm.at[pl.ds(sid * RPT, RPT)],
                    deg_sh.at[pl.ds(sid * RPT, RPT)])
    pltpu.sync_copy(er_hbm.at[1, wid], dst2)
    pltpu.sync_copy(ones_hbm, ones_v)
    plsc.subcore_barrier()
    for i in range(NCHUNK):
        pltpu.sync_copy(ones_v, deg_sh.at[dst2.at[i]], add=True)
    plsc.subcore_barrier()
    pltpu.sync_copy(deg_sh.at[pl.ds(sid * RPT, RPT)],
                    out_hbm.at[cid, pl.ds(sid * RPT, RPT)])


NBUF = 3


def _agg_body(g_hbm, er_hbm, zeros_hbm, out_hbm,
              agg_sh, src2, dst2, rows0, rows1, rows2, sem0, sem1, sem2):
    cid = lax.axis_index("c")
    sid = lax.axis_index("s")
    wid = sid * NC + cid
    bufs = [(rows0, sem0), (rows1, sem1), (rows2, sem2)]
    pltpu.sync_copy(zeros_hbm.at[pl.ds(sid * RPT, RPT), :],
                    agg_sh.at[pl.ds(sid * RPT, RPT), :])
    pltpu.sync_copy(er_hbm.at[0, wid], src2)
    pltpu.sync_copy(er_hbm.at[1, wid], dst2)
    plsc.subcore_barrier()
    for k in range(NBUF):
        rows, sem = bufs[k]
        pltpu.async_copy(g_hbm.at[src2.at[k]], rows, sem)
    for k in range(NCHUNK):
        rows, sem = bufs[k % NBUF]
        pltpu.make_async_copy(g_hbm.at[src2.at[k]], rows, sem).wait()
        pltpu.sync_copy(rows, agg_sh.at[dst2.at[k]], add=True)
        if k + NBUF < NCHUNK:
            pltpu.async_copy(g_hbm.at[src2.at[k + NBUF]], rows, sem)
    plsc.subcore_barrier()
    pltpu.sync_copy(agg_sh.at[pl.ds(sid * RPT, RPT), :],
                    out_hbm.at[cid, pl.ds(sid * RPT, RPT), :])


import functools


@functools.cache
def _sc_calls():
    mesh = plsc.VectorSubcoreMesh(
        core_axis_name="c", subcore_axis_name="s",
        num_cores=NC, num_subcores=NS,
    )
    params = pltpu.CompilerParams(use_tc_tiling_on_sc=False)
    deg_call = pl.kernel(
        _deg_body,
        out_type=jax.ShapeDtypeStruct((NC, NP), jnp.float32),
        mesh=mesh,
        compiler_params=params,
        scratch_types=[
            pltpu.VMEM_SHARED((NP,), jnp.float32),
            pltpu.VMEM((NCHUNK, EB), jnp.int32),
            pltpu.VMEM((EB,), jnp.float32),
        ],
    )
    agg_call = pl.kernel(
        _agg_body,
        out_type=jax.ShapeDtypeStruct((NC, NP, H), jnp.float32),
        mesh=mesh,
        compiler_params=params,
        scratch_types=[
            pltpu.VMEM_SHARED((NP, H), jnp.float32),
            pltpu.VMEM((NCHUNK, EB), jnp.int32),
            pltpu.VMEM((NCHUNK, EB), jnp.int32),
            pltpu.VMEM((EB, H), jnp.float32),
            pltpu.VMEM((EB, H), jnp.float32),
            pltpu.VMEM((EB, H), jnp.float32),
            pltpu.SemaphoreType.DMA,
            pltpu.SemaphoreType.DMA,
            pltpu.SemaphoreType.DMA,
        ],
    )
    return deg_call, agg_call


_PACK = 128 // H


def _unpack(vp):
    r = vp.shape[0]
    t = jnp.broadcast_to(vp[:, None, :], (r, _PACK, 128))
    t = jnp.reshape(t, (_PACK * r, 128))
    n = lax.broadcasted_iota(jnp.int32, (_PACK * r, 1), 0) % _PACK
    out = jnp.where(n == 0, 1.0, 0.0) * t[:, 0:H]
    for j in range(1, _PACK):
        out = out + jnp.where(n == j, 1.0, 0.0) * t[:, H * j:H * (j + 1)]
    return out


def _pack(v):
    r4 = v.shape[0]
    wide = jnp.concatenate([v] * _PACK, axis=1)
    c = lax.broadcasted_iota(jnp.int32, (r4, 128), 1) // H
    n = lax.broadcasted_iota(jnp.int32, (r4, 128), 0) % _PACK
    wide = jnp.where(c == n, wide, 0.0)
    return jnp.sum(jnp.reshape(wide, (r4 // _PACK, _PACK, 128)), axis=1)


def _col_from_rows(v128, nrows):
    r = v128.shape[0]
    t = jnp.broadcast_to(v128[:, None, :], (r, 128, 128))
    t = jnp.reshape(t, (128 * r, 128))
    m = (lax.broadcasted_iota(jnp.int32, (128 * r, 128), 1)
         == lax.broadcasted_iota(jnp.int32, (128 * r, 128), 0) % 128)
    return jnp.sum(jnp.where(m, t, 0.0), axis=1, keepdims=True)[:nrows]


def _tcr_body(ei_ref, out_ref):
    v = ei_ref[...]
    sr = jnp.reshape(v[0], (E // 128, 128))
    dr = jnp.reshape(v[1], (E // 128, 128))
    out_ref[...] = jnp.concatenate(
        [jnp.reshape(sr, (1, E // 128, 128)),
         jnp.reshape(dr, (1, E // 128, 128))], axis=0)


_tcr = pl.pallas_call(
    _tcr_body,
    out_shape=jax.ShapeDtypeStruct((2, E // 128, 128), jnp.int32),
)


_NPK = N * H // 128


def _blockdiag4(w):
    z = jnp.zeros((H, H), jnp.float32)
    rows = []
    for j in range(_PACK):
        rows.append(jnp.concatenate(
            [w if k == j else z for k in range(_PACK)], axis=1))
    return jnp.concatenate(rows, axis=0)


def _tile_row(b):
    return jnp.concatenate([b] * _PACK, axis=1)


def _tc1_body(x_ref, w1_ref, degp_ref, g1p_ref, dinvp_ref):
    v = degp_ref[...]
    deg = v[0] + v[1] + 1.0
    dinv = _col_from_rows(lax.rsqrt(deg), N)
    dinvp = _pack(jnp.broadcast_to(dinv, (N, H)))
    dinvp_ref[...] = dinvp
    p1 = jnp.dot(x_ref[...], w1_ref[...], preferred_element_type=jnp.float32)
    g1p_ref[...] = _pack(p1) * dinvp


_tc1 = pl.pallas_call(
    _tc1_body,
    out_shape=(jax.ShapeDtypeStruct((_NPK, 128), jnp.float32),
               jax.ShapeDtypeStruct((_NPK, 128), jnp.float32)),
)


def _tc2_body(aggp_ref, g1p_ref, dinvp_ref, b1_ref, w2_ref, g2p_ref):
    va = aggp_ref[...]
    dinvp = dinvp_ref[...]
    z = ((va[0, :_NPK] + va[1, :_NPK] + g1p_ref[...]) * dinvp
         + _tile_row(b1_ref[...]))
    h1p = jnp.maximum(z, 0.0)
    g2p_ref[...] = jnp.dot(h1p, _blockdiag4(w2_ref[...]),
                           preferred_element_type=jnp.float32) * dinvp


_tc2 = pl.pallas_call(
    _tc2_body,
    out_shape=jax.ShapeDtypeStruct((_NPK, 128), jnp.float32),
)


def _tc3_body(aggp_ref, g2p_ref, dinvp_ref, b2_ref, batch_ref, wl_ref, bl_ref,
              out_ref):
    va = aggp_ref[...]
    z = ((va[0, :_NPK] + va[1, :_NPK] + g2p_ref[...]) * dinvp_ref[...]
         + _tile_row(b2_ref[...]))
    h2 = _unpack(jnp.maximum(z, 0.0))
    bt = batch_ref[...]
    gid = lax.broadcasted_iota(jnp.int32, (G, N), 0)
    oh = (gid == bt).astype(jnp.float32)
    sums = jnp.dot(oh, h2, preferred_element_type=jnp.float32)
    cnt = jnp.sum(oh, axis=1, keepdims=True)
    pooled = sums / jnp.maximum(cnt, 1.0)
    out_ref[...] = jnp.dot(pooled, wl_ref[...],
                           preferred_element_type=jnp.float32) + bl_ref[...]


_tc3 = pl.pallas_call(
    _tc3_body,
    out_shape=jax.ShapeDtypeStruct((G, 2), jnp.float32),
)


def kernel(x, edge_index, batch, W1, b1, W2, b2, Wl, bl):
    _deg_call, _agg_call = _sc_calls()
    er4 = _tcr(edge_index.astype(jnp.int32)).reshape(2, NW, NCHUNK, EB)
    zeros1 = jnp.zeros((NP,), jnp.float32)
    zeros2 = jnp.zeros((NP, H), jnp.float32)
    onesb = jnp.ones((EB,), jnp.float32)
    degp = _deg_call(er4, zeros1, onesb)
    g1p, dinvp = _tc1(x, W1, degp.reshape(NC, NP // 128, 128))
    g1 = g1p.reshape(N, H)
    aggp1 = _agg_call(g1, er4, zeros2)
    g2p = _tc2(aggp1.reshape(NC, NP * H // 128, 128), g1p, dinvp,
               b1.reshape(1, H), W2)
    g2 = g2p.reshape(N, H)
    aggp2 = _agg_call(g2, er4, zeros2)
    return _tc3(aggp2.reshape(NC, NP * H // 128, 128), g2p, dinvp,
                b2.reshape(1, H), batch.reshape(1, N),
                Wl, bl.reshape(1, 2))


# speedup vs baseline: 1.1021x; 1.1021x over previous
import jax
import jax.numpy as jnp
from jax import lax
from jax.experimental import pallas as pl
from jax.experimental.pallas import tpu as pltpu
from jax.experimental.pallas import tpu_sc as plsc

N = 10000
E = 320000
D = 128
H = 32
G = 64

NC = 2
NS = 16
NW = NC * NS
EPW = E // NW
EB = 1000
NCHUNK = EPW // EB
NP = 10240
RPT = NP // NS

def _deg_body(er_hbm, zeros_hbm, ones_hbm, out_hbm, deg_sh, dst2, ones_v):
    cid = lax.axis_index("c")
    sid = lax.axis_index("s")
    wid = sid * NC + cid
    pltpu.sync_copy(zeros_hbm.at[pl.ds(sid * RPT, RPT)],
                    deg_sh.at[pl.ds(sid * RPT, RPT)])
    pltpu.sync_copy(er_hbm.at[1, wid], dst2)
    pltpu.sync_copy(ones_hbm, ones_v)
    plsc.subcore_barrier()
    for i in range(NCHUNK):
        pltpu.sync_copy(ones_v, deg_sh.at[dst2.at[i]], add=True)
    plsc.subcore_barrier()
    pltpu.sync_copy(deg_sh.at[pl.ds(sid * RPT, RPT)],
                    out_hbm.at[cid, pl.ds(sid * RPT, RPT)])


NBUF = 2


def _agg_body(g_hbm, er_hbm, zeros_hbm, out_hbm,
              agg_sh, src2, dst2, rows0, rows1, sem0, sem1):
    cid = lax.axis_index("c")
    sid = lax.axis_index("s")
    wid = sid * NC + cid
    bufs = [(rows0, sem0), (rows1, sem1)]
    pltpu.sync_copy(zeros_hbm.at[pl.ds(sid * RPT, RPT), :],
                    agg_sh.at[pl.ds(sid * RPT, RPT), :])
    pltpu.sync_copy(er_hbm.at[0, wid], src2)
    pltpu.sync_copy(er_hbm.at[1, wid], dst2)
    plsc.subcore_barrier()
    for k in range(NBUF):
        rows, sem = bufs[k]
        pltpu.async_copy(g_hbm.at[src2.at[k]], rows, sem)
    for k in range(NCHUNK):
        rows, sem = bufs[k % NBUF]
        pltpu.make_async_copy(g_hbm.at[src2.at[k]], rows, sem).wait()
        pltpu.sync_copy(rows, agg_sh.at[dst2.at[k]], add=True)
        if k + NBUF < NCHUNK:
            pltpu.async_copy(g_hbm.at[src2.at[k + NBUF]], rows, sem)
    plsc.subcore_barrier()
    pltpu.sync_copy(agg_sh.at[pl.ds(sid * RPT, RPT), :],
                    out_hbm.at[cid, pl.ds(sid * RPT, RPT), :])


import functools


@functools.cache
def _sc_calls():
    mesh = plsc.VectorSubcoreMesh(
        core_axis_name="c", subcore_axis_name="s",
        num_cores=NC, num_subcores=NS,
    )
    params = pltpu.CompilerParams(use_tc_tiling_on_sc=False)
    deg_call = pl.kernel(
        _deg_body,
        out_type=jax.ShapeDtypeStruct((NC, NP), jnp.float32),
        mesh=mesh,
        compiler_params=params,
        scratch_types=[
            pltpu.VMEM_SHARED((NP,), jnp.float32),
            pltpu.VMEM((NCHUNK, EB), jnp.int32),
            pltpu.VMEM((EB,), jnp.float32),
        ],
    )
    agg_call = pl.kernel(
        _agg_body,
        out_type=jax.ShapeDtypeStruct((NC, NP, H), jnp.float32),
        mesh=mesh,
        compiler_params=params,
        scratch_types=[
            pltpu.VMEM_SHARED((NP, H), jnp.float32),
            pltpu.VMEM((NCHUNK, EB), jnp.int32),
            pltpu.VMEM((NCHUNK, EB), jnp.int32),
            pltpu.VMEM((EB, H), jnp.float32),
            pltpu.VMEM((EB, H), jnp.float32),
            pltpu.SemaphoreType.DMA,
            pltpu.SemaphoreType.DMA,
        ],
    )
    return deg_call, agg_call


_PACK = 128 // H


def _unpack(vp):
    r = vp.shape[0]
    t = jnp.broadcast_to(vp[:, None, :], (r, _PACK, 128))
    t = jnp.reshape(t, (_PACK * r, 128))
    n = lax.broadcasted_iota(jnp.int32, (_PACK * r, 1), 0) % _PACK
    out = jnp.where(n == 0, 1.0, 0.0) * t[:, 0:H]
    for j in range(1, _PACK):
        out = out + jnp.where(n == j, 1.0, 0.0) * t[:, H * j:H * (j + 1)]
    return out


def _pack(v):
    r4 = v.shape[0]
    wide = jnp.concatenate([v] * _PACK, axis=1)
    c = lax.broadcasted_iota(jnp.int32, (r4, 128), 1) // H
    n = lax.broadcasted_iota(jnp.int32, (r4, 128), 0) % _PACK
    wide = jnp.where(c == n, wide, 0.0)
    return jnp.sum(jnp.reshape(wide, (r4 // _PACK, _PACK, 128)), axis=1)


def _col_from_rows(v128, nrows):
    r = v128.shape[0]
    t = jnp.broadcast_to(v128[:, None, :], (r, 128, 128))
    t = jnp.reshape(t, (128 * r, 128))
    m = (lax.broadcasted_iota(jnp.int32, (128 * r, 128), 1)
         == lax.broadcasted_iota(jnp.int32, (128 * r, 128), 0) % 128)
    return jnp.sum(jnp.where(m, t, 0.0), axis=1, keepdims=True)[:nrows]


def _tcr_body(ei_ref, out_ref):
    v = ei_ref[...]
    sr = jnp.reshape(v[0], (E // 128, 128))
    dr = jnp.reshape(v[1], (E // 128, 128))
    out_ref[...] = jnp.concatenate(
        [jnp.reshape(sr, (1, E // 128, 128)),
         jnp.reshape(dr, (1, E // 128, 128))], axis=0)


_tcr = pl.pallas_call(
    _tcr_body,
    out_shape=jax.ShapeDtypeStruct((2, E // 128, 128), jnp.int32),
)


_NPK = N * H // 128


def _blockdiag4(w):
    z = jnp.zeros((H, H), jnp.float32)
    rows = []
    for j in range(_PACK):
        rows.append(jnp.concatenate(
            [w if k == j else z for k in range(_PACK)], axis=1))
    return jnp.concatenate(rows, axis=0)


def _tile_row(b):
    return jnp.concatenate([b] * _PACK, axis=1)


def _tc1a_body(x_ref, w1_ref, p1p_ref):
    p1 = jnp.dot(x_ref[...], w1_ref[...], preferred_element_type=jnp.float32)
    p1p_ref[...] = _pack(p1)


_tc1a = pl.pallas_call(
    _tc1a_body,
    out_shape=jax.ShapeDtypeStruct((_NPK, 128), jnp.float32),
)


def _tc1b_body(p1p_ref, degp_ref, g1p_ref, dinvp_ref):
    v = degp_ref[...]
    deg = v[0] + v[1] + 1.0
    dinv = _col_from_rows(lax.rsqrt(deg), N)
    dinvp = _pack(jnp.broadcast_to(dinv, (N, H)))
    dinvp_ref[...] = dinvp
    g1p_ref[...] = p1p_ref[...] * dinvp


_tc1b = pl.pallas_call(
    _tc1b_body,
    out_shape=(jax.ShapeDtypeStruct((_NPK, 128), jnp.float32),
               jax.ShapeDtypeStruct((_NPK, 128), jnp.float32)),
)


def _tc2_body(aggp_ref, g1p_ref, dinvp_ref, b1_ref, w2_ref, g2p_ref):
    va = aggp_ref[...]
    dinvp = dinvp_ref[...]
    z = ((va[0, :_NPK] + va[1, :_NPK] + g1p_ref[...]) * dinvp
         + _tile_row(b1_ref[...]))
    h1p = jnp.maximum(z, 0.0)
    g2p_ref[...] = jnp.dot(h1p, _blockdiag4(w2_ref[...]),
                           preferred_element_type=jnp.float32) * dinvp


_tc2 = pl.pallas_call(
    _tc2_body,
    out_shape=jax.ShapeDtypeStruct((_NPK, 128), jnp.float32),
)


def _tc3_body(aggp_ref, g2p_ref, dinvp_ref, b2_ref, batch_ref, wl_ref, bl_ref,
              out_ref):
    va = aggp_ref[...]
    z = ((va[0, :_NPK] + va[1, :_NPK] + g2p_ref[...]) * dinvp_ref[...]
         + _tile_row(b2_ref[...]))
    h2 = _unpack(jnp.maximum(z, 0.0))
    bt = batch_ref[...]
    gid = lax.broadcasted_iota(jnp.int32, (G, N), 0)
    oh = (gid == bt).astype(jnp.float32)
    sums = jnp.dot(oh, h2, preferred_element_type=jnp.float32)
    cnt = jnp.sum(oh, axis=1, keepdims=True)
    pooled = sums / jnp.maximum(cnt, 1.0)
    out_ref[...] = jnp.dot(pooled, wl_ref[...],
                           preferred_element_type=jnp.float32) + bl_ref[...]


_tc3 = pl.pallas_call(
    _tc3_body,
    out_shape=jax.ShapeDtypeStruct((G, 2), jnp.float32),
)


def kernel(x, edge_index, batch, W1, b1, W2, b2, Wl, bl):
    _deg_call, _agg_call = _sc_calls()
    er4 = _tcr(edge_index.astype(jnp.int32)).reshape(2, NW, NCHUNK, EB)
    zeros1 = jnp.zeros((NP,), jnp.float32)
    zeros2 = jnp.zeros((NP, H), jnp.float32)
    onesb = jnp.ones((EB,), jnp.float32)
    degp = _deg_call(er4, zeros1, onesb)
    p1p = _tc1a(x, W1)
    g1p, dinvp = _tc1b(p1p, degp.reshape(NC, NP // 128, 128))
    g1 = g1p.reshape(N, H)
    aggp1 = _agg_call(g1, er4, zeros2)
    g2p = _tc2(aggp1.reshape(NC, NP * H // 128, 128), g1p, dinvp,
               b1.reshape(1, H), W2)
    g2 = g2p.reshape(N, H)
    aggp2 = _agg_call(g2, er4, zeros2)
    return _tc3(aggp2.reshape(NC, NP * H // 128, 128), g2p, dinvp,
                b2.reshape(1, H), batch.reshape(1, N),
                Wl, bl.reshape(1, 2))
